# Initial kernel scaffold; baseline (speedup 1.0000x reference)
import jax
import jax.numpy as jnp
from jax.experimental import pallas as pl


def kernel(token_embeddings, attention_mask, index_keys):
    raise NotImplementedError("write your pallas kernel here")


# speedup vs baseline: 1.6067x; 1.6067x over previous
import functools
import math

import jax
import jax.numpy as jnp
from jax.experimental import pallas as pl
from jax.experimental.pallas import tpu as pltpu

_TOP_K = 5
_NEG_INF = float("-inf")
_BIG_I32 = 2**31 - 1


def _embed_body(tok_ref, mask_ref, out_ref, *, d_out):
    t = tok_ref[...]
    m = mask_ref[...]
    summed = jnp.sum(t * m[:, :, None], axis=1)
    counts = jnp.clip(jnp.sum(m, axis=1, keepdims=True), 1e-9, None)
    e = summed / counts
    mu = jnp.mean(e, axis=1, keepdims=True)
    var = jnp.mean((e - mu) ** 2, axis=1, keepdims=True)
    e = (e - mu) / jnp.sqrt(var + 1e-5)
    e = e[:, :d_out]
    nrm = jnp.sqrt(jnp.sum(e * e, axis=1, keepdims=True))
    out_ref[...] = e / jnp.maximum(nrm, 1e-12)


def _topk_body(emb_ref, keys_ref, vals_ref, idx_ref, cv_ref, cg_ref,
               *, nk, bk, kblocks):
    k = pl.program_id(0)

    @pl.when(k == 0)
    def _init():
        cv_ref[...] = jnp.full(cv_ref.shape, _NEG_INF, jnp.float32)
        cg_ref[...] = jnp.full(cg_ref.shape, _BIG_I32, jnp.int32)

    keys = keys_ref[...]
    n2 = jnp.sum(keys * keys, axis=1, keepdims=True)
    keys_n = keys / jnp.maximum(jnp.sqrt(n2), 1e-12)
    emb = emb_ref[...]
    scores = jax.lax.dot_general(
        emb, keys_n, (((1,), (1,)), ((), ())),
        preferred_element_type=jnp.float32)
    bsz = scores.shape[0]
    gcol = jax.lax.broadcasted_iota(jnp.int32, (bsz, bk), 1) + k * bk
    s = jnp.where(gcol < nk, scores, _NEG_INF)

    vs, gs = [], []
    for _ in range(_TOP_K):
        m = jnp.max(s, axis=1, keepdims=True)
        g = jnp.min(jnp.where(s == m, gcol, _BIG_I32), axis=1, keepdims=True)
        vs.append(m)
        gs.append(g)
        s = jnp.where(gcol == g, _NEG_INF, s)
    nv = jnp.concatenate(vs, axis=1)
    ng = jnp.concatenate(gs, axis=1)

    cand_v = jnp.concatenate([cv_ref[...], nv], axis=1)
    cand_g = jnp.concatenate([cg_ref[...], ng], axis=1)
    mv, mg = [], []
    for _ in range(_TOP_K):
        m = jnp.max(cand_v, axis=1, keepdims=True)
        g = jnp.min(jnp.where(cand_v == m, cand_g, _BIG_I32),
                    axis=1, keepdims=True)
        mv.append(m)
        mg.append(g)
        cand_v = jnp.where(cand_g == g, _NEG_INF, cand_v)
    merged_v = jnp.concatenate(mv, axis=1)
    merged_g = jnp.concatenate(mg, axis=1)
    cv_ref[...] = merged_v
    cg_ref[...] = merged_g

    @pl.when(k == kblocks - 1)
    def _out():
        vals_ref[...] = merged_v
        idx_ref[...] = merged_g


def kernel(token_embeddings, attention_mask, index_keys):
    B, T, H = token_embeddings.shape
    NK, D = index_keys.shape

    bb = 64 if B % 64 == 0 else B
    emb = pl.pallas_call(
        functools.partial(_embed_body, d_out=D),
        grid=(B // bb,),
        in_specs=[
            pl.BlockSpec((bb, T, H), lambda i: (i, 0, 0)),
            pl.BlockSpec((bb, T), lambda i: (i, 0)),
        ],
        out_specs=pl.BlockSpec((bb, D), lambda i: (i, 0)),
        out_shape=jax.ShapeDtypeStruct((B, D), jnp.float32),
    )(token_embeddings, attention_mask)

    bk = 2048
    kblocks = math.ceil(NK / bk)
    vals, idx = pl.pallas_call(
        functools.partial(_topk_body, nk=NK, bk=bk, kblocks=kblocks),
        grid=(kblocks,),
        in_specs=[
            pl.BlockSpec((B, D), lambda k: (0, 0)),
            pl.BlockSpec((bk, D), lambda k: (k, 0)),
        ],
        out_specs=[
            pl.BlockSpec((B, _TOP_K), lambda k: (0, 0)),
            pl.BlockSpec((B, _TOP_K), lambda k: (0, 0)),
        ],
        out_shape=[
            jax.ShapeDtypeStruct((B, _TOP_K), jnp.float32),
            jax.ShapeDtypeStruct((B, _TOP_K), jnp.int32),
        ],
        scratch_shapes=[
            pltpu.VMEM((B, _TOP_K), jnp.float32),
            pltpu.VMEM((B, _TOP_K), jnp.int32),
        ],
        compiler_params=pltpu.CompilerParams(
            dimension_semantics=("arbitrary",),
        ),
    )(emb, index_keys)
    return vals, idx


# speedup vs baseline: 2.4485x; 1.5240x over previous
import functools
import math

import jax
import jax.numpy as jnp
from jax.experimental import pallas as pl
from jax.experimental.pallas import tpu as pltpu

_TOP_K = 5
_NCAND = 16
_SCALE = 8192.0
_IDX_BITS = 17
_IDX_MAX = (1 << _IDX_BITS) - 1
_NEG_INF = float("-inf")
_BIG_I32 = 2**31 - 1
_MIN_I32 = -(2**31)


def _embed_body(tok_ref, mask_ref, out_ref, *, d_out):
    t = tok_ref[...]
    m = mask_ref[...]
    summed = jnp.sum(t * m[:, :, None], axis=1)
    counts = jnp.clip(jnp.sum(m, axis=1, keepdims=True), 1e-9, None)
    e = summed / counts
    mu = jnp.mean(e, axis=1, keepdims=True)
    var = jnp.mean((e - mu) ** 2, axis=1, keepdims=True)
    e = (e - mu) / jnp.sqrt(var + 1e-5)
    e = e[:, :d_out]
    nrm = jnp.sqrt(jnp.sum(e * e, axis=1, keepdims=True))
    out_ref[...] = e / jnp.maximum(nrm, 1e-12) * _SCALE


def _scan_body(emb_ref, keys_ref, col_ref, cand_ref, carry_ref,
               *, nk, bk, kblocks):
    k = pl.program_id(0)

    @pl.when(k == 0)
    def _init():
        carry_ref[...] = jnp.full(carry_ref.shape, _MIN_I32, jnp.int32)

    keys = keys_ref[...]
    ssq = jnp.sum(keys * keys, axis=1)
    rn = 1.0 / jnp.maximum(jnp.sqrt(ssq), 1e-12)
    emb = emb_ref[...]
    s = jax.lax.dot_general(
        emb, keys, (((1,), (1,)), ((), ())),
        preferred_element_type=jnp.float32)
    s = s * rn[None, :]
    gcol = col_ref[0]
    s = jnp.where(gcol < nk, s, -2.0 * _SCALE)
    q = s.astype(jnp.int32)
    p = q * (_IDX_MAX + 1) + (_IDX_MAX - gcol)

    m1 = jnp.full((p.shape[0], 128), _MIN_I32, jnp.int32)
    m2 = m1
    for t in range(bk // 128):
        pt = p[:, t * 128:(t + 1) * 128]
        lo = jnp.minimum(m1, pt)
        m1 = jnp.maximum(m1, pt)
        m2 = jnp.maximum(m2, lo)

    c1, c2, c3, c4 = (carry_ref[0], carry_ref[1],
                      carry_ref[2], carry_ref[3])
    for mnew in (m1, m2):
        e1 = jnp.minimum(c1, mnew)
        c1 = jnp.maximum(c1, mnew)
        e2 = jnp.minimum(c2, e1)
        c2 = jnp.maximum(c2, e1)
        e3 = jnp.minimum(c3, e2)
        c3 = jnp.maximum(c3, e2)
        c4 = jnp.maximum(c4, e3)
    carry_ref[0], carry_ref[1] = c1, c2
    carry_ref[2], carry_ref[3] = c3, c4

    @pl.when(k == kblocks - 1)
    def _extract():
        c = jnp.concatenate([c1, c2, c3, c4], axis=1)
        picks = []
        for _ in range(_NCAND):
            m = jnp.max(c, axis=1, keepdims=True)
            picks.append(m)
            c = jnp.where(c == m, _MIN_I32, c)
        packed = jnp.concatenate(picks, axis=1)
        cand_ref[...] = _IDX_MAX - (packed & _IDX_MAX)


def _rescore_body(emb_ref, rows_ref, gidx_ref, vals_ref, idx_ref, *, nk):
    rows = rows_ref[...]
    nrm = jnp.sqrt(jnp.sum(rows * rows, axis=1, keepdims=True))
    rows_n = rows / jnp.clip(nrm, 1e-12, None)
    e = emb_ref[...] * (1.0 / _SCALE)
    s = jax.lax.dot_general(
        e, rows_n, (((1,), (1,)), ((), ())),
        preferred_element_type=jnp.float32)
    g = gidx_ref[0]
    col = jax.lax.broadcasted_iota(jnp.int32, s.shape, 1)
    row = jax.lax.broadcasted_iota(jnp.int32, s.shape, 0)
    own = (col // _NCAND == row) & (g < nk)
    s = jnp.where(own, s, _NEG_INF)
    vs, gs = [], []
    for _ in range(_TOP_K):
        m = jnp.max(s, axis=1, keepdims=True)
        sel = jnp.min(jnp.where(s == m, g, _BIG_I32),
                      axis=1, keepdims=True)
        vs.append(m)
        gs.append(sel)
        s = jnp.where(g == sel, _NEG_INF, s)
    vals_ref[...] = jnp.concatenate(vs, axis=1)
    idx_ref[...] = jnp.concatenate(gs, axis=1)


def kernel(token_embeddings, attention_mask, index_keys):
    B, T, H = token_embeddings.shape
    NK, D = index_keys.shape

    bb = 64 if B % 64 == 0 else B
    emb = pl.pallas_call(
        functools.partial(_embed_body, d_out=D),
        grid=(B // bb,),
        in_specs=[
            pl.BlockSpec((bb, T, H), lambda i: (i, 0, 0)),
            pl.BlockSpec((bb, T), lambda i: (i, 0)),
        ],
        out_specs=pl.BlockSpec((bb, D), lambda i: (i, 0)),
        out_shape=jax.ShapeDtypeStruct((B, D), jnp.float32),
    )(token_embeddings, attention_mask)

    bk = 2048
    kblocks = math.ceil(NK / bk)
    cols = jnp.arange(kblocks * bk, dtype=jnp.int32).reshape(kblocks, 1, bk)
    cand = pl.pallas_call(
        functools.partial(_scan_body, nk=NK, bk=bk, kblocks=kblocks),
        grid=(kblocks,),
        in_specs=[
            pl.BlockSpec((B, D), lambda k: (0, 0)),
            pl.BlockSpec((bk, D), lambda k: (k, 0)),
            pl.BlockSpec((1, 1, bk), lambda k: (k, 0, 0)),
        ],
        out_specs=pl.BlockSpec((B, _NCAND), lambda k: (0, 0)),
        out_shape=jax.ShapeDtypeStruct((B, _NCAND), jnp.int32),
        scratch_shapes=[
            pltpu.VMEM((4, B, 128), jnp.int32),
        ],
        compiler_params=pltpu.CompilerParams(
            dimension_semantics=("arbitrary",),
        ),
    )(emb, index_keys, cols)

    safe = jnp.minimum(cand, NK - 1)
    rows = jnp.take(index_keys, safe.reshape(-1), axis=0)

    bq = 128 if B % 128 == 0 else B
    cand_flat = cand.reshape(B // bq, 1, bq * _NCAND)
    vals, idx = pl.pallas_call(
        functools.partial(_rescore_body, nk=NK),
        grid=(B // bq,),
        in_specs=[
            pl.BlockSpec((bq, D), lambda i: (i, 0)),
            pl.BlockSpec((bq * _NCAND, D), lambda i: (i, 0)),
            pl.BlockSpec((1, 1, bq * _NCAND), lambda i: (i, 0, 0)),
        ],
        out_specs=[
            pl.BlockSpec((bq, _TOP_K), lambda i: (i, 0)),
            pl.BlockSpec((bq, _TOP_K), lambda i: (i, 0)),
        ],
        out_shape=[
            jax.ShapeDtypeStruct((B, _TOP_K), jnp.float32),
            jax.ShapeDtypeStruct((B, _TOP_K), jnp.int32),
        ],
    )(emb, rows, cand_flat)
    return vals, idx


# speedup vs baseline: 2.4639x; 1.0063x over previous
import functools
import math

import jax
import jax.numpy as jnp
from jax.experimental import pallas as pl
from jax.experimental.pallas import tpu as pltpu

_TOP_K = 5
_NCAND = 16
_SCALE = 8192.0
_IDX_BITS = 17
_IDX_MAX = (1 << _IDX_BITS) - 1
_NEG_INF = float("-inf")
_BIG_I32 = 2**31 - 1
_MIN_I32 = -(2**31)


def _embed_body(tok_ref, mask_ref, out_ref, *, d_out):
    t = tok_ref[...]
    m = mask_ref[...]
    summed = jnp.sum(t * m[:, :, None], axis=1)
    counts = jnp.clip(jnp.sum(m, axis=1, keepdims=True), 1e-9, None)
    e = summed / counts
    mu = jnp.mean(e, axis=1, keepdims=True)
    var = jnp.mean((e - mu) ** 2, axis=1, keepdims=True)
    e = (e - mu) / jnp.sqrt(var + 1e-5)
    e = e[:, :d_out]
    nrm = jnp.sqrt(jnp.sum(e * e, axis=1, keepdims=True))
    out_ref[...] = e / jnp.maximum(nrm, 1e-12) * _SCALE


def _scan_body(emb_ref, keys_ref, col_ref, cand_ref, carry_ref,
               *, nk, bk, kblocks):
    k = pl.program_id(0)

    @pl.when(k == 0)
    def _init():
        carry_ref[...] = jnp.full(carry_ref.shape, _MIN_I32, jnp.int32)

    keys = keys_ref[...]
    ssq = jnp.sum(keys * keys, axis=1)
    rn = 1.0 / jnp.maximum(jnp.sqrt(ssq), 1e-12)
    emb = emb_ref[...]
    s = jax.lax.dot_general(
        emb.astype(jnp.bfloat16), keys.astype(jnp.bfloat16),
        (((1,), (1,)), ((), ())),
        preferred_element_type=jnp.float32)
    s = s * rn[None, :]
    gcol = col_ref[0]
    s = jnp.where(gcol < nk, s, -2.0 * _SCALE)
    q = s.astype(jnp.int32)
    p = q * (_IDX_MAX + 1) + (_IDX_MAX - gcol)

    m1 = jnp.full((p.shape[0], 128), _MIN_I32, jnp.int32)
    m2 = m1
    for t in range(bk // 128):
        pt = p[:, t * 128:(t + 1) * 128]
        lo = jnp.minimum(m1, pt)
        m1 = jnp.maximum(m1, pt)
        m2 = jnp.maximum(m2, lo)

    c1, c2, c3, c4 = (carry_ref[0], carry_ref[1],
                      carry_ref[2], carry_ref[3])
    for mnew in (m1, m2):
        e1 = jnp.minimum(c1, mnew)
        c1 = jnp.maximum(c1, mnew)
        e2 = jnp.minimum(c2, e1)
        c2 = jnp.maximum(c2, e1)
        e3 = jnp.minimum(c3, e2)
        c3 = jnp.maximum(c3, e2)
        c4 = jnp.maximum(c4, e3)
    carry_ref[0], carry_ref[1] = c1, c2
    carry_ref[2], carry_ref[3] = c3, c4

    @pl.when(k == kblocks - 1)
    def _extract():
        c = jnp.concatenate([c1, c2, c3, c4], axis=1)
        picks = []
        for _ in range(_NCAND):
            m = jnp.max(c, axis=1, keepdims=True)
            picks.append(m)
            c = jnp.where(c == m, _MIN_I32, c)
        packed = jnp.concatenate(picks, axis=1)
        cand_ref[...] = _IDX_MAX - (packed & _IDX_MAX)


def _rescore_body(emb_ref, rows_ref, gidx_ref, vals_ref, idx_ref, *, nk):
    rows = rows_ref[...]
    nrm = jnp.sqrt(jnp.sum(rows * rows, axis=1, keepdims=True))
    rows_n = rows / jnp.clip(nrm, 1e-12, None)
    e = emb_ref[...] * (1.0 / _SCALE)
    s = jax.lax.dot_general(
        e, rows_n, (((1,), (1,)), ((), ())),
        preferred_element_type=jnp.float32)
    g = gidx_ref[0]
    col = jax.lax.broadcasted_iota(jnp.int32, s.shape, 1)
    row = jax.lax.broadcasted_iota(jnp.int32, s.shape, 0)
    own = (col // _NCAND == row) & (g < nk)
    s = jnp.where(own, s, _NEG_INF)
    vs, gs = [], []
    for _ in range(_TOP_K):
        m = jnp.max(s, axis=1, keepdims=True)
        sel = jnp.min(jnp.where(s == m, g, _BIG_I32),
                      axis=1, keepdims=True)
        vs.append(m)
        gs.append(sel)
        s = jnp.where(g == sel, _NEG_INF, s)
    vals_ref[...] = jnp.concatenate(vs, axis=1)
    idx_ref[...] = jnp.concatenate(gs, axis=1)


def kernel(token_embeddings, attention_mask, index_keys):
    B, T, H = token_embeddings.shape
    NK, D = index_keys.shape

    bb = 64 if B % 64 == 0 else B
    emb = pl.pallas_call(
        functools.partial(_embed_body, d_out=D),
        grid=(B // bb,),
        in_specs=[
            pl.BlockSpec((bb, T, H), lambda i: (i, 0, 0)),
            pl.BlockSpec((bb, T), lambda i: (i, 0)),
        ],
        out_specs=pl.BlockSpec((bb, D), lambda i: (i, 0)),
        out_shape=jax.ShapeDtypeStruct((B, D), jnp.float32),
    )(token_embeddings, attention_mask)

    bk = 2048
    kblocks = math.ceil(NK / bk)
    cols = jnp.arange(kblocks * bk, dtype=jnp.int32).reshape(kblocks, 1, bk)
    cand = pl.pallas_call(
        functools.partial(_scan_body, nk=NK, bk=bk, kblocks=kblocks),
        grid=(kblocks,),
        in_specs=[
            pl.BlockSpec((B, D), lambda k: (0, 0)),
            pl.BlockSpec((bk, D), lambda k: (k, 0)),
            pl.BlockSpec((1, 1, bk), lambda k: (k, 0, 0)),
        ],
        out_specs=pl.BlockSpec((B, _NCAND), lambda k: (0, 0)),
        out_shape=jax.ShapeDtypeStruct((B, _NCAND), jnp.int32),
        scratch_shapes=[
            pltpu.VMEM((4, B, 128), jnp.int32),
        ],
        compiler_params=pltpu.CompilerParams(
            dimension_semantics=("arbitrary",),
        ),
    )(emb, index_keys, cols)

    safe = jnp.minimum(cand, NK - 1)
    rows = jnp.take(index_keys, safe.reshape(-1), axis=0)

    bq = 128 if B % 128 == 0 else B
    cand_flat = cand.reshape(B // bq, 1, bq * _NCAND)
    vals, idx = pl.pallas_call(
        functools.partial(_rescore_body, nk=NK),
        grid=(B // bq,),
        in_specs=[
            pl.BlockSpec((bq, D), lambda i: (i, 0)),
            pl.BlockSpec((bq * _NCAND, D), lambda i: (i, 0)),
            pl.BlockSpec((1, 1, bq * _NCAND), lambda i: (i, 0, 0)),
        ],
        out_specs=[
            pl.BlockSpec((bq, _TOP_K), lambda i: (i, 0)),
            pl.BlockSpec((bq, _TOP_K), lambda i: (i, 0)),
        ],
        out_shape=[
            jax.ShapeDtypeStruct((B, _TOP_K), jnp.float32),
            jax.ShapeDtypeStruct((B, _TOP_K), jnp.int32),
        ],
    )(emb, rows, cand_flat)
    return vals, idx
